# baseline (speedup 1.0000x reference)
import functools

import jax
import jax.numpy as jnp
from jax import lax
from jax.experimental import pallas as pl
from jax.experimental.pallas import tpu as pltpu
from jax.experimental.pallas import tpu_sc as plsc

DIM = 64
NCH = 8
NDG = DIM // 16
CB = 64
_IN_BOUNDS = lax.GatherScatterMode.PROMISE_IN_BOUNDS


def _take(vec, idx):
    return vec.at[idx].get(mode=_IN_BOUNDS)


@functools.lru_cache(maxsize=None)
def _make(B: int, VOCAB: int):
    info = plsc.get_sparse_core_info()
    NC, NS = info.num_cores, info.num_subcores
    NW = NC * NS
    BPW = B // NW
    NCHUNK = BPW // CB
    scale = float(DIM ** 0.5 * NCH ** -0.5)

    mesh = plsc.VectorSubcoreMesh(core_axis_name="c", subcore_axis_name="s")

    @functools.partial(
        pl.kernel,
        mesh=mesh,
        out_type=jax.ShapeDtypeStruct((B, DIM), jnp.float32),
        compiler_params=pltpu.CompilerParams(use_tc_tiling_on_sc=False),
        scratch_types=[
            pltpu.VMEM((BPW,), jnp.int32),
            pltpu.VMEM((NCH, BPW), jnp.int32),
            pltpu.VMEM((NCH, BPW), jnp.int32),
            pltpu.VMEM((NCH, BPW), jnp.int32),
            pltpu.VMEM((NCH, BPW), jnp.float32),
            pltpu.VMEM((2, NCH, CB, DIM), jnp.float32),
            pltpu.VMEM((CB, DIM), jnp.float32),
            pltpu.SemaphoreType.DMA,
            pltpu.SemaphoreType.DMA,
            pltpu.SemaphoreType.DMA,
        ],
    )
    def k(x_hbm, tab_hbm, flat_hbm, h0f_hbm, h1f_hbm,
          out_hbm,
          x_v, ix_v, i0c_v, i1c_v, w_v, rows_v, o_v,
          semA, semB, semW):
        sem_par = [semA, semB]
        wid = lax.axis_index("s") * NC + lax.axis_index("c")
        base = wid * BPW

        pltpu.sync_copy(x_hbm.at[pl.ds(base, BPW)], x_v)

        def mkoff(g, carry):
            xg = x_v[pl.ds(g * 16, 16)]
            for c in range(NCH):
                ix_v[c, pl.ds(g * 16, 16)] = xg + c * VOCAB
            return carry

        lax.fori_loop(0, BPW // 16, mkoff, 0)

        descs = []
        for j in range(BPW // 128):
            for c in range(NCH):
                off = ix_v.at[c, pl.ds(j * 128, 128)]
                descs.append(pltpu.async_copy(
                    h0f_hbm.at[off], i0c_v.at[c, pl.ds(j * 128, 128)], semW))
                descs.append(pltpu.async_copy(
                    h1f_hbm.at[off], i1c_v.at[c, pl.ds(j * 128, 128)], semW))
        for d in descs:
            d.wait()

        wdescs = []
        for c in range(NCH):
            for j in range(BPW // 128):
                wdescs.append(pltpu.async_copy(
                    flat_hbm.at[i1c_v.at[c, pl.ds(j * 128, 128)]],
                    w_v.at[c, pl.ds(j * 128, 128)], semW))

        def issue_rows(chunk):
            par = chunk % 2
            return [pltpu.async_copy(
                tab_hbm.at[i0c_v.at[c, pl.ds(chunk * CB, CB)]],
                rows_v.at[par, c], sem_par[par]) for c in range(NCH)]

        pending = issue_rows(0)
        for d in wdescs:
            d.wait()

        for chunk in range(NCHUNK):
            par = chunk % 2
            for d in pending:
                d.wait()
            if chunk + 1 < NCHUNK:
                pending = issue_rows(chunk + 1)

            def body(b, carry):
                bsplat = jnp.broadcast_to(lax.bitwise_and(b, 15), (16,))
                bg16 = lax.shift_right_logical(b, 4) * 16
                acc = [None] * NDG
                for c in range(NCH):
                    wrow = w_v[c, pl.ds(chunk * CB + bg16, 16)]
                    wsplat = _take(wrow, bsplat) * scale
                    for q in range(NDG):
                        term = wsplat * rows_v[par, c, b, pl.ds(q * 16, 16)]
                        acc[q] = term if c == 0 else acc[q] + term
                for q in range(NDG):
                    o_v[b, pl.ds(q * 16, 16)] = acc[q]
                return carry

            lax.fori_loop(0, CB, body, 0)
            pltpu.sync_copy(o_v, out_hbm.at[pl.ds(base + chunk * CB, CB)])

    return k


def _force_flat(a):
    f = a.reshape(-1)
    return f.at[0].set(f[0])


def kernel(x, table, h0, h1):
    B = x.shape[0]
    h0f = _force_flat(h0.astype(jnp.int32).T)
    h1f = _force_flat(h1.astype(jnp.int32).T)
    flat = _force_flat(table)
    return _make(B, h0.shape[0])(x.astype(jnp.int32), table, flat, h0f, h1f)


# speedup vs baseline: 3.3771x; 3.3771x over previous
import functools

import jax
import jax.numpy as jnp
from jax import lax
from jax.experimental import pallas as pl
from jax.experimental.pallas import tpu as pltpu
from jax.experimental.pallas import tpu_sc as plsc

DIM = 64
NCH = 8
NDG = DIM // 16
CB = 64
_IN_BOUNDS = lax.GatherScatterMode.PROMISE_IN_BOUNDS


def _take(vec, idx):
    return vec.at[idx].get(mode=_IN_BOUNDS)


@functools.lru_cache(maxsize=None)
def _make(B: int):
    info = plsc.get_sparse_core_info()
    NC, NS = info.num_cores, info.num_subcores
    NW = NC * NS
    BPW = B // NW
    NCHUNK = BPW // CB
    scale = float(DIM ** 0.5 * NCH ** -0.5)

    mesh = plsc.VectorSubcoreMesh(core_axis_name="c", subcore_axis_name="s")

    @functools.partial(
        pl.kernel,
        mesh=mesh,
        out_type=jax.ShapeDtypeStruct((B, DIM), jnp.float32),
        compiler_params=pltpu.CompilerParams(use_tc_tiling_on_sc=False),
        scratch_types=[
            pltpu.VMEM((BPW,), jnp.int32),
            pltpu.VMEM((NCH, BPW), jnp.int32),
            pltpu.VMEM((NCH, BPW), jnp.int32),
            pltpu.VMEM((NCH, BPW), jnp.float32),
            pltpu.VMEM((2, NCH, CB, DIM), jnp.float32),
            pltpu.VMEM((CB, DIM), jnp.float32),
            pltpu.SemaphoreType.DMA,
            pltpu.SemaphoreType.DMA,
            pltpu.SemaphoreType.DMA,
        ],
    )
    def k(x_hbm, tab_hbm, flat_hbm,
          h0c0, h0c1, h0c2, h0c3, h0c4, h0c5, h0c6, h0c7,
          h1c0, h1c1, h1c2, h1c3, h1c4, h1c5, h1c6, h1c7,
          out_hbm,
          x_v, i0c_v, i1c_v, w_v, rows_v, o_v,
          semA, semB, semW):
        h0c = [h0c0, h0c1, h0c2, h0c3, h0c4, h0c5, h0c6, h0c7]
        h1c = [h1c0, h1c1, h1c2, h1c3, h1c4, h1c5, h1c6, h1c7]
        sem_par = [semA, semB]
        wid = lax.axis_index("s") * NC + lax.axis_index("c")
        base = wid * BPW

        pltpu.sync_copy(x_hbm.at[pl.ds(base, BPW)], x_v)

        descs = []
        for j in range(BPW // 128):
            off = x_v.at[pl.ds(j * 128, 128)]
            for c in range(NCH):
                descs.append(pltpu.async_copy(
                    h0c[c].at[off], i0c_v.at[c, pl.ds(j * 128, 128)], semW))
                descs.append(pltpu.async_copy(
                    h1c[c].at[off], i1c_v.at[c, pl.ds(j * 128, 128)], semW))
        for d in descs:
            d.wait()

        wdescs = []
        for c in range(NCH):
            for j in range(BPW // 128):
                wdescs.append(pltpu.async_copy(
                    flat_hbm.at[i1c_v.at[c, pl.ds(j * 128, 128)]],
                    w_v.at[c, pl.ds(j * 128, 128)], semW))

        def issue_rows(chunk):
            par = chunk % 2
            return [pltpu.async_copy(
                tab_hbm.at[i0c_v.at[c, pl.ds(chunk * CB, CB)]],
                rows_v.at[par, c], sem_par[par]) for c in range(NCH)]

        pending = issue_rows(0)
        for d in wdescs:
            d.wait()

        for chunk in range(NCHUNK):
            par = chunk % 2
            for d in pending:
                d.wait()
            if chunk + 1 < NCHUNK:
                pending = issue_rows(chunk + 1)

            def body(b, carry):
                bsplat = jnp.broadcast_to(lax.bitwise_and(b, 15), (16,))
                bg16 = lax.shift_right_logical(b, 4) * 16
                acc = [None] * NDG
                for c in range(NCH):
                    wrow = w_v[c, pl.ds(chunk * CB + bg16, 16)]
                    wsplat = _take(wrow, bsplat) * scale
                    for q in range(NDG):
                        term = wsplat * rows_v[par, c, b, pl.ds(q * 16, 16)]
                        acc[q] = term if c == 0 else acc[q] + term
                for q in range(NDG):
                    o_v[b, pl.ds(q * 16, 16)] = acc[q]
                return carry

            lax.fori_loop(0, CB, body, 0)
            pltpu.sync_copy(o_v, out_hbm.at[pl.ds(base + chunk * CB, CB)])

    return k


def kernel(x, table, h0, h1):
    B = x.shape[0]
    h0 = h0.astype(jnp.int32)
    h1 = h1.astype(jnp.int32)
    cols = [h0[:, c] for c in range(NCH)] + [h1[:, c] for c in range(NCH)]
    flat = table.reshape(-1)
    flat = flat.at[0].set(flat[0])
    return _make(B)(x.astype(jnp.int32), table, flat, *cols)
